# Initial kernel scaffold; baseline (speedup 1.0000x reference)
import jax
import jax.numpy as jnp
from jax.experimental import pallas as pl


def kernel(lhs_idx, rhs_idx, emb, rel_vec):
    raise NotImplementedError("write your pallas kernel here")


# speedup vs baseline: 1.0498x; 1.0498x over previous
import functools

import jax
import jax.numpy as jnp
from jax import lax
from jax.experimental import pallas as pl
from jax.experimental.pallas import tpu as pltpu
from jax.experimental.pallas import tpu_sc as plsc

DIM = 128
CHUNK = 512
IDX_CHUNK = 128


def _gather_body(nw, b_per_w, n_idx_chunks, d,
                 lhs_idx_hbm, rhs_idx_hbm, table_hbm,
                 lhs_out, rhs_out, idx_v, rows_v, sem):
  info = plsc.get_sparse_core_info()
  wid = lax.axis_index("s") * info.num_cores + lax.axis_index("c")
  base = wid * b_per_w
  for src_idx, dst in ((lhs_idx_hbm, lhs_out), (rhs_idx_hbm, rhs_out)):
    pltpu.sync_copy(src_idx.at[wid], idx_v)
    for j in range(n_idx_chunks):
      pltpu.async_copy(table_hbm.at[idx_v.at[j]],
                       rows_v.at[pl.ds(j * IDX_CHUNK, IDX_CHUNK)], sem).wait()
    pltpu.sync_copy(rows_v, dst.at[pl.ds(base, b_per_w)])


def _sc_gather(lhs_idx, rhs_idx, emb):
  vocab, d = emb.shape
  b = lhs_idx.shape[0]
  info = plsc.get_sparse_core_info()
  nw = info.num_cores * info.num_subcores
  b_per_w = b // nw
  n_idx_chunks = b_per_w // IDX_CHUNK
  mesh = plsc.VectorSubcoreMesh(core_axis_name="c", subcore_axis_name="s")
  kern = functools.partial(
      pl.kernel,
      mesh=mesh,
      out_type=[
          jax.ShapeDtypeStruct((b, d), jnp.float32),
          jax.ShapeDtypeStruct((b, d), jnp.float32),
      ],
      scratch_types=[
          pltpu.VMEM((n_idx_chunks, IDX_CHUNK), jnp.int32),
          pltpu.VMEM((b_per_w, d), jnp.float32),
          pltpu.SemaphoreType.DMA,
      ],
  )(functools.partial(_gather_body, nw, b_per_w, n_idx_chunks, d))
  lhs_idx_3d = lhs_idx.reshape(nw, n_idx_chunks, IDX_CHUNK).astype(jnp.int32)
  rhs_idx_3d = rhs_idx.reshape(nw, n_idx_chunks, IDX_CHUNK).astype(jnp.int32)
  return kern(lhs_idx_3d, rhs_idx_3d, emb)


def _score_body(lhs_ref, rhs_ref, rel_ref, pos_ref, ln_ref, rn_ref):
  lhs = lhs_ref[0]
  rhs = rhs_ref[0] * rel_ref[...]
  dn = (((1,), (1,)), ((), ()))
  s = lax.dot_general(rhs, lhs, dn, preferred_element_type=jnp.float32)
  st = lax.dot_general(lhs, rhs, dn, preferred_element_type=jnp.float32)
  pos_ref[0] = jnp.sum(lhs_ref[...] * (rhs_ref[...] * rel_ref[...][None]),
                       axis=2)
  r = lax.broadcasted_iota(jnp.int32, (CHUNK, CHUNK), 0)
  c = lax.broadcasted_iota(jnp.int32, (CHUNK, CHUNK), 1)
  eye = r == c
  neg = jnp.float32(-1e9)
  ln_ref[0] = jnp.where(eye, neg, s)
  rn_ref[0] = jnp.where(eye, neg, st)


def _tc_score(lhs_g, rhs_g, rel_vec):
  b, d = lhs_g.shape
  c = b // CHUNK
  lhs_c = lhs_g.reshape(c, CHUNK, d)
  rhs_c = rhs_g.reshape(c, CHUNK, d)
  rel2 = rel_vec.reshape(1, d)
  return pl.pallas_call(
      _score_body,
      grid=(c,),
      in_specs=[
          pl.BlockSpec((1, CHUNK, d), lambda i: (i, 0, 0)),
          pl.BlockSpec((1, CHUNK, d), lambda i: (i, 0, 0)),
          pl.BlockSpec((1, d), lambda i: (0, 0)),
      ],
      out_specs=[
          pl.BlockSpec((1, 1, CHUNK), lambda i: (i, 0, 0)),
          pl.BlockSpec((1, CHUNK, CHUNK), lambda i: (i, 0, 0)),
          pl.BlockSpec((1, CHUNK, CHUNK), lambda i: (i, 0, 0)),
      ],
      out_shape=[
          jax.ShapeDtypeStruct((c, 1, CHUNK), jnp.float32),
          jax.ShapeDtypeStruct((c, CHUNK, CHUNK), jnp.float32),
          jax.ShapeDtypeStruct((c, CHUNK, CHUNK), jnp.float32),
      ],
  )(lhs_c, rhs_c, rel2)


def _tc_score_outputs(lhs_g, rhs_g, rel_vec):
  pos3, ln, rn = _tc_score(lhs_g, rhs_g, rel_vec)
  return pos3.reshape(pos3.shape[0], pos3.shape[2]), ln, rn


def kernel(lhs_idx, rhs_idx, emb, rel_vec):
  lhs_g, rhs_g = _sc_gather(lhs_idx, rhs_idx, emb)
  pos, ln, rn = _tc_score_outputs(lhs_g, rhs_g, rel_vec)
  return pos, ln, rn


# speedup vs baseline: 1.0994x; 1.0472x over previous
import functools

import jax
import jax.numpy as jnp
from jax import lax
from jax.experimental import pallas as pl
from jax.experimental.pallas import tpu as pltpu
from jax.experimental.pallas import tpu_sc as plsc

DIM = 128
CHUNK = 512
IDX_CHUNK = 128


_NBUF = 2


def _gather_body(nw, b_per_w, n_idx_chunks, d,
                 lhs_idx_hbm, rhs_idx_hbm, table_hbm,
                 lhs_out, rhs_out, idx_v, rows_v,
                 gsem0, gsem1, wsem0, wsem1):
  info = plsc.get_sparse_core_info()
  wid = lax.axis_index("s") * info.num_cores + lax.axis_index("c")
  base = wid * b_per_w
  pltpu.sync_copy(lhs_idx_hbm.at[wid], idx_v.at[0])
  pltpu.sync_copy(rhs_idx_hbm.at[wid], idx_v.at[1])
  gsems = (gsem0, gsem1)
  wsems = (wsem0, wsem1)
  ntasks = 2 * n_idx_chunks
  tasks = [(side, j) for side in range(2) for j in range(n_idx_chunks)]
  outs = (lhs_out, rhs_out)
  gh = [None] * ntasks
  wh = [None] * ntasks
  for t in range(ntasks + 1):
    if t < ntasks:
      side, j = tasks[t]
      b = t % _NBUF
      if t >= _NBUF:
        wh[t - _NBUF].wait()
      gh[t] = pltpu.async_copy(table_hbm.at[idx_v.at[side, j]],
                               rows_v.at[b], gsems[b])
    if t >= 1:
      side, j = tasks[t - 1]
      b = (t - 1) % _NBUF
      gh[t - 1].wait()
      wh[t - 1] = pltpu.async_copy(
          rows_v.at[b],
          outs[side].at[pl.ds(base + j * IDX_CHUNK, IDX_CHUNK)], wsems[b])
  wh[ntasks - 2].wait()
  wh[ntasks - 1].wait()


def _sc_gather(lhs_idx, rhs_idx, emb):
  vocab, d = emb.shape
  b = lhs_idx.shape[0]
  info = plsc.get_sparse_core_info()
  nw = info.num_cores * info.num_subcores
  b_per_w = b // nw
  n_idx_chunks = b_per_w // IDX_CHUNK
  mesh = plsc.VectorSubcoreMesh(core_axis_name="c", subcore_axis_name="s")
  kern = functools.partial(
      pl.kernel,
      mesh=mesh,
      out_type=[
          jax.ShapeDtypeStruct((b, d), jnp.float32),
          jax.ShapeDtypeStruct((b, d), jnp.float32),
      ],
      scratch_types=[
          pltpu.VMEM((2, n_idx_chunks, IDX_CHUNK), jnp.int32),
          pltpu.VMEM((_NBUF, IDX_CHUNK, d), jnp.float32),
          pltpu.SemaphoreType.DMA,
          pltpu.SemaphoreType.DMA,
          pltpu.SemaphoreType.DMA,
          pltpu.SemaphoreType.DMA,
      ],
  )(functools.partial(_gather_body, nw, b_per_w, n_idx_chunks, d))
  lhs_idx_3d = lhs_idx.reshape(nw, n_idx_chunks, IDX_CHUNK).astype(jnp.int32)
  rhs_idx_3d = rhs_idx.reshape(nw, n_idx_chunks, IDX_CHUNK).astype(jnp.int32)
  return kern(lhs_idx_3d, rhs_idx_3d, emb)


def _score_body(lhs_ref, rhs_ref, rel_ref, pos_ref, ln_ref, rn_ref):
  lhs = lhs_ref[0]
  rhs = rhs_ref[0] * rel_ref[...]
  dn = (((1,), (1,)), ((), ()))
  s = lax.dot_general(rhs, lhs, dn, preferred_element_type=jnp.float32)
  st = lax.dot_general(lhs, rhs, dn, preferred_element_type=jnp.float32)
  pos_ref[0] = jnp.sum(lhs_ref[...] * (rhs_ref[...] * rel_ref[...][None]),
                       axis=2)
  r = lax.broadcasted_iota(jnp.int32, (CHUNK, CHUNK), 0)
  c = lax.broadcasted_iota(jnp.int32, (CHUNK, CHUNK), 1)
  eye = r == c
  neg = jnp.float32(-1e9)
  ln_ref[0] = jnp.where(eye, neg, s)
  rn_ref[0] = jnp.where(eye, neg, st)


def _tc_score(lhs_g, rhs_g, rel_vec):
  b, d = lhs_g.shape
  c = b // CHUNK
  lhs_c = lhs_g.reshape(c, CHUNK, d)
  rhs_c = rhs_g.reshape(c, CHUNK, d)
  rel2 = rel_vec.reshape(1, d)
  return pl.pallas_call(
      _score_body,
      grid=(c,),
      in_specs=[
          pl.BlockSpec((1, CHUNK, d), lambda i: (i, 0, 0)),
          pl.BlockSpec((1, CHUNK, d), lambda i: (i, 0, 0)),
          pl.BlockSpec((1, d), lambda i: (0, 0)),
      ],
      out_specs=[
          pl.BlockSpec((1, 1, CHUNK), lambda i: (i, 0, 0)),
          pl.BlockSpec((1, CHUNK, CHUNK), lambda i: (i, 0, 0)),
          pl.BlockSpec((1, CHUNK, CHUNK), lambda i: (i, 0, 0)),
      ],
      out_shape=[
          jax.ShapeDtypeStruct((c, 1, CHUNK), jnp.float32),
          jax.ShapeDtypeStruct((c, CHUNK, CHUNK), jnp.float32),
          jax.ShapeDtypeStruct((c, CHUNK, CHUNK), jnp.float32),
      ],
  )(lhs_c, rhs_c, rel2)


def _tc_score_outputs(lhs_g, rhs_g, rel_vec):
  pos3, ln, rn = _tc_score(lhs_g, rhs_g, rel_vec)
  return pos3.reshape(pos3.shape[0], pos3.shape[2]), ln, rn


def kernel(lhs_idx, rhs_idx, emb, rel_vec):
  lhs_g, rhs_g = _sc_gather(lhs_idx, rhs_idx, emb)
  pos, ln, rn = _tc_score_outputs(lhs_g, rhs_g, rel_vec)
  return pos, ln, rn


# speedup vs baseline: 1.1018x; 1.0022x over previous
import functools

import jax
import jax.numpy as jnp
from jax import lax
from jax.experimental import pallas as pl
from jax.experimental.pallas import tpu as pltpu
from jax.experimental.pallas import tpu_sc as plsc

DIM = 128
CHUNK = 512
IDX_CHUNK = 128


_NBUF = 2


def _gather_body(nw, b_per_w, n_idx_chunks, d,
                 lhs_idx_hbm, rhs_idx_hbm, table_hbm,
                 lhs_out, rhs_out, idx_v, rows_v,
                 gsem0, gsem1, wsem0, wsem1):
  info = plsc.get_sparse_core_info()
  wid = lax.axis_index("s") * info.num_cores + lax.axis_index("c")
  base = wid * b_per_w
  pltpu.sync_copy(lhs_idx_hbm.at[wid], idx_v.at[0])
  pltpu.sync_copy(rhs_idx_hbm.at[wid], idx_v.at[1])
  gsems = (gsem0, gsem1)
  wsems = (wsem0, wsem1)
  ntasks = 2 * n_idx_chunks
  tasks = [(side, j) for side in range(2) for j in range(n_idx_chunks)]
  outs = (lhs_out, rhs_out)
  gh = [None] * ntasks
  wh = [None] * ntasks
  for t in range(ntasks + 1):
    if t < ntasks:
      side, j = tasks[t]
      b = t % _NBUF
      if t >= _NBUF:
        wh[t - _NBUF].wait()
      gh[t] = pltpu.async_copy(table_hbm.at[idx_v.at[side, j]],
                               rows_v.at[b], gsems[b])
    if t >= 1:
      side, j = tasks[t - 1]
      b = (t - 1) % _NBUF
      gh[t - 1].wait()
      wh[t - 1] = pltpu.async_copy(
          rows_v.at[b],
          outs[side].at[pl.ds(base + j * IDX_CHUNK, IDX_CHUNK)], wsems[b])
  wh[ntasks - 2].wait()
  wh[ntasks - 1].wait()


def _sc_gather(lhs_idx, rhs_idx, emb):
  vocab, d = emb.shape
  b = lhs_idx.shape[0]
  info = plsc.get_sparse_core_info()
  nw = info.num_cores * info.num_subcores
  b_per_w = b // nw
  n_idx_chunks = b_per_w // IDX_CHUNK
  mesh = plsc.VectorSubcoreMesh(core_axis_name="c", subcore_axis_name="s")
  kern = functools.partial(
      pl.kernel,
      mesh=mesh,
      out_type=[
          jax.ShapeDtypeStruct((b, d), jnp.float32),
          jax.ShapeDtypeStruct((b, d), jnp.float32),
      ],
      scratch_types=[
          pltpu.VMEM((2, n_idx_chunks, IDX_CHUNK), jnp.int32),
          pltpu.VMEM((_NBUF, IDX_CHUNK, d), jnp.float32),
          pltpu.SemaphoreType.DMA,
          pltpu.SemaphoreType.DMA,
          pltpu.SemaphoreType.DMA,
          pltpu.SemaphoreType.DMA,
      ],
  )(functools.partial(_gather_body, nw, b_per_w, n_idx_chunks, d))
  lhs_idx_3d = lhs_idx.reshape(nw, n_idx_chunks, IDX_CHUNK).astype(jnp.int32)
  rhs_idx_3d = rhs_idx.reshape(nw, n_idx_chunks, IDX_CHUNK).astype(jnp.int32)
  return kern(lhs_idx_3d, rhs_idx_3d, emb)


def _score_body(*refs):
  lhs_ref, rhs_ref, rel_ref = refs[0], refs[1], refs[2]
  pos_ref, ln_ref, rn_ref = refs[-3], refs[-2], refs[-1]
  lhs = lhs_ref[0]
  rhs = rhs_ref[0] * rel_ref[...]
  dn = (((1,), (1,)), ((), ()))
  s = lax.dot_general(rhs, lhs, dn, preferred_element_type=jnp.float32)
  st = lax.dot_general(lhs, rhs, dn, preferred_element_type=jnp.float32)
  pos_ref[0] = jnp.sum(lhs_ref[...] * (rhs_ref[...] * rel_ref[...][None]),
                       axis=2)
  r = lax.broadcasted_iota(jnp.int32, (CHUNK, CHUNK), 0)
  c = lax.broadcasted_iota(jnp.int32, (CHUNK, CHUNK), 1)
  eye = r == c
  neg = jnp.float32(-1e9)
  ln_ref[0] = jnp.where(eye, neg, s)
  rn_ref[0] = jnp.where(eye, neg, st)


def _tc_score_part(lhs_g, rhs_g, rel_vec, c_off, c_total, prev):
  b, d = lhs_g.shape
  cp = b // CHUNK
  lhs_c = lhs_g.reshape(cp, CHUNK, d)
  rhs_c = rhs_g.reshape(cp, CHUNK, d)
  rel2 = rel_vec.reshape(1, d)
  in_specs = [
      pl.BlockSpec((1, CHUNK, d), lambda i: (i, 0, 0)),
      pl.BlockSpec((1, CHUNK, d), lambda i: (i, 0, 0)),
      pl.BlockSpec((1, d), lambda i: (0, 0)),
  ]
  args = [lhs_c, rhs_c, rel2]
  aliases = {}
  if prev is not None:
    for k in range(3):
      in_specs.append(pl.BlockSpec(memory_space=pl.ANY))
      args.append(prev[k])
      aliases[3 + k] = k
  return pl.pallas_call(
      _score_body,
      grid=(cp,),
      in_specs=in_specs,
      out_specs=[
          pl.BlockSpec((1, 1, CHUNK), lambda i: (i + c_off, 0, 0)),
          pl.BlockSpec((1, CHUNK, CHUNK), lambda i: (i + c_off, 0, 0)),
          pl.BlockSpec((1, CHUNK, CHUNK), lambda i: (i + c_off, 0, 0)),
      ],
      out_shape=[
          jax.ShapeDtypeStruct((c_total, 1, CHUNK), jnp.float32),
          jax.ShapeDtypeStruct((c_total, CHUNK, CHUNK), jnp.float32),
          jax.ShapeDtypeStruct((c_total, CHUNK, CHUNK), jnp.float32),
      ],
      input_output_aliases=aliases,
  )(*args)


_NPARTS = 2


def kernel(lhs_idx, rhs_idx, emb, rel_vec):
  b = lhs_idx.shape[0]
  c_total = b // CHUNK
  bp = b // _NPARTS
  c_off = c_total // _NPARTS
  gathered = [
      _sc_gather(lhs_idx[p * bp:(p + 1) * bp],
                 rhs_idx[p * bp:(p + 1) * bp], emb)
      for p in range(_NPARTS)
  ]
  prev = None
  for p in range(_NPARTS):
    prev = _tc_score_part(gathered[p][0], gathered[p][1], rel_vec,
                          p * c_off, c_total, prev)
  pos3, ln, rn = prev
  return pos3.reshape(c_total, CHUNK), ln, rn
